# baseline (speedup 1.0000x reference)
import functools

import numpy as np
import jax
import jax.numpy as jnp
from jax import lax
from jax.experimental import pallas as pl
from jax.experimental.pallas import tpu as pltpu
from jax.experimental.pallas import tpu_sc as plsc

N_ROWS = 16384
N_COLS = 1000
PAD = 0
SMOOTH = 0.1
CONF = 1.0 - SMOOTH
EPS = SMOOTH / (N_COLS - 2)
C_ROW = float(SMOOTH * np.log(EPS) + CONF * np.log(CONF))

NC = 2
NS = 16
L = 16
NW = NC * NS

SC_COLS = 4096
TC_COLS = N_ROWS - SC_COLS
COLS_PER_W = SC_COLS // NW
BC = 128
NBATCH = COLS_PER_W // BC
GPB = BC // L
SLAB = 192
TAILSLAB = N_COLS - 5 * SLAB
RSTEP = 8

TCW = 512
TC_NBLK = TC_COLS // TCW
TGT_ROWS = TCW // 128


def _sc_body(xt_hbm, t_hbm, out_hbm, buf0, buf1, tbuf, gbuf, ovec,
             sem0, sem1, semg):
    cid = lax.axis_index("c")
    sid = lax.axis_index("s")
    wid = sid * NC + cid
    col0 = wid * COLS_PER_W

    pltpu.sync_copy(t_hbm.at[pl.ds(col0, COLS_PER_W)], tbuf)

    zeros = jnp.zeros((L,), jnp.float32)
    ones = jnp.full((L,), 1.0, jnp.float32)
    iota = lax.iota(jnp.int32, L)

    def dma(b, r0, nr, buf, sem):
        cstart = pl.multiple_of(col0 + b * BC, BC)
        return pltpu.make_async_copy(
            xt_hbm.at[pl.ds(r0, nr), pl.ds(cstart, BC)],
            buf.at[pl.ds(0, nr)], sem)

    def group_meta(b, g):
        toff = pl.multiple_of(b * BC + g * L, L)
        tg = tbuf[pl.ds(toff, L)]
        validf = lax.select(tg != PAD, ones, zeros)
        return tg, validf

    def proc(buf, nrows, rbase, b, total, accs):
        first = jnp.where(rbase == 0, 1.0, 0.0)
        new_accs = []
        for g in range(GPB):
            tg, validf = group_meta(b, g)

            def dense_step(j, a4, g=g):
                a0, a1, a2, a3 = a4
                r = j * RSTEP
                s = [buf[r + u, pl.ds(g * L, L)] for u in range(RSTEP)]
                return ((a0 + s[0]) + s[4], (a1 + s[1]) + s[5],
                        (a2 + s[2]) + s[6], (a3 + s[3]) + s[7])

            a0, a1, a2, a3 = lax.fori_loop(
                0, nrows // RSTEP, dense_step, (zeros, zeros, zeros, zeros))
            new_accs.append(accs[g] + ((a0 + a1) + (a2 + a3)))

            x00 = buf[0, pl.ds(g * L, L)]
            total = total + first * (validf * (C_ROW + EPS * x00))
        return total, new_accs

    dma(0, 0, SLAB, buf0, sem0).start()

    def batch_body(b, total):
        accs = [zeros] * GPB

        cstart = pl.multiple_of(col0 + b * BC, BC)
        tidx = tbuf.at[pl.ds(pl.multiple_of(b * BC, BC), BC)]
        gather = pltpu.make_async_copy(
            xt_hbm.at[tidx, pl.ds(cstart, BC)], gbuf, semg)
        gather.start()

        def pair_body(p, carry):
            total, *accs = carry
            r0 = p * (2 * SLAB)
            dma(b, r0 + SLAB, SLAB, buf1, sem1).start()
            dma(b, r0, SLAB, buf0, sem0).wait()
            total, accs = proc(buf0, SLAB, r0, b, total, accs)
            dma(b, r0 + 2 * SLAB, SLAB, buf0, sem0).start()
            dma(b, r0 + SLAB, SLAB, buf1, sem1).wait()
            total, accs = proc(buf1, SLAB, r0 + SLAB, b, total, accs)
            return (total, *accs)

        total, *accs = lax.fori_loop(0, 2, pair_body, (total, *accs))

        dma(b, 5 * SLAB, TAILSLAB, buf1, sem1).start()
        dma(b, 4 * SLAB, SLAB, buf0, sem0).wait()
        total, accs = proc(buf0, SLAB, 4 * SLAB, b, total, accs)

        @pl.when(b + 1 < NBATCH)
        def _prefetch():
            dma(b + 1, 0, SLAB, buf0, sem0).start()

        dma(b, 5 * SLAB, TAILSLAB, buf1, sem1).wait()
        total, accs = proc(buf1, TAILSLAB, 5 * SLAB, b, total, accs)

        gather.wait()
        for g in range(GPB):
            _, validf = group_meta(b, g)
            xt_g = zeros
            for l in range(L):
                v = gbuf[g * L + l, pl.ds(g * L, L)]
                xt_g = lax.select(iota == l, v, xt_g)
            total = total - EPS * (validf * accs[g]) \
                - (CONF - EPS) * (validf * xt_g)
        return total

    total = lax.fori_loop(0, NBATCH, batch_body, zeros)
    ovec[...] = total
    pltpu.sync_copy(ovec, out_hbm.at[wid])


@functools.partial(
    pl.kernel,
    mesh=plsc.VectorSubcoreMesh(core_axis_name="c", subcore_axis_name="s"),
    compiler_params=pltpu.CompilerParams(skip_device_barrier=True),
    out_type=jax.ShapeDtypeStruct((NW, L), jnp.float32),
    scratch_types=[
        pltpu.VMEM((SLAB, BC), jnp.float32),
        pltpu.VMEM((SLAB, BC), jnp.float32),
        pltpu.VMEM((COLS_PER_W,), jnp.int32),
        pltpu.VMEM((BC, BC), jnp.float32),
        pltpu.VMEM((L,), jnp.float32),
        pltpu.SemaphoreType.DMA,
        pltpu.SemaphoreType.DMA,
        pltpu.SemaphoreType.DMA,
    ],
)
def _sc_kernel(xt_hbm, t_hbm, out_hbm, buf0, buf1, tbuf, gbuf, ovec,
               sem0, sem1, semg):
    _sc_body(xt_hbm, t_hbm, out_hbm, buf0, buf1, tbuf, gbuf, ovec,
             sem0, sem1, semg)


def _tc_body(xt_ref, tgt_ref, out_ref):
    i = pl.program_id(0)
    x = xt_ref[...]
    t = tgt_ref[...].reshape(1, TCW)
    riota = lax.broadcasted_iota(jnp.int32, (N_COLS, TCW), 0)
    xt_sel = jnp.where(riota == t, x, 0.0)

    colsum = jnp.sum(x, axis=0, keepdims=True)
    xt_val = jnp.sum(xt_sel, axis=0, keepdims=True)
    x0 = x[0:1, :]
    valid = jnp.where(t != PAD, 1.0, 0.0)
    per_col = valid * (C_ROW + EPS * x0 - (CONF - EPS) * xt_val
                       - EPS * colsum)
    part = jnp.sum(per_col.reshape(TGT_ROWS, 128), axis=0, keepdims=True)

    @pl.when(i == 0)
    def _init():
        out_ref[...] = jnp.zeros_like(out_ref)

    out_ref[...] = out_ref[...] + part


_tc_kernel = pl.pallas_call(
    _tc_body,
    grid=(TC_NBLK,),
    in_specs=[
        pl.BlockSpec((N_COLS, TCW), lambda i: (0, SC_COLS // TCW + i)),
        pl.BlockSpec((TCW,), lambda i: (SC_COLS // TCW + i,)),
    ],
    out_specs=pl.BlockSpec((1, 128), lambda i: (0, 0)),
    out_shape=jax.ShapeDtypeStruct((1, 128), jnp.float32),
    compiler_params=pltpu.CompilerParams(
        dimension_semantics=("arbitrary",),
    ),
)


def kernel(x, target):
    xt = x.T
    tgt = target.astype(jnp.int32)
    sc_part = _sc_kernel(xt, tgt)
    tc_part = _tc_kernel(xt, tgt)
    return jnp.sum(sc_part) + jnp.sum(tc_part)


# speedup vs baseline: 1.1006x; 1.1006x over previous
import functools

import numpy as np
import jax
import jax.numpy as jnp
from jax import lax
from jax.experimental import pallas as pl
from jax.experimental.pallas import tpu as pltpu
from jax.experimental.pallas import tpu_sc as plsc

N_ROWS = 16384
N_COLS = 1000
PAD = 0
SMOOTH = 0.1
CONF = 1.0 - SMOOTH
EPS = SMOOTH / (N_COLS - 2)
C_ROW = float(SMOOTH * np.log(EPS) + CONF * np.log(CONF))

NC = 2
NS = 16
L = 16
NW = NC * NS

SC_COLS = 4096
TC_COLS = N_ROWS - SC_COLS
COLS_PER_W = SC_COLS // NW
BC = 128
NBATCH = COLS_PER_W // BC
GPB = BC // L
SLAB = 192
TAILSLAB = N_COLS - 5 * SLAB
RSTEP = 8

TCW = 1024
TC_NBLK = TC_COLS // TCW
TGT_ROWS = TCW // 128


def _sc_body(xt_hbm, t_hbm, out_hbm, buf0, buf1, tbuf, gbuf, ovec,
             sem0, sem1, semg):
    cid = lax.axis_index("c")
    sid = lax.axis_index("s")
    wid = sid * NC + cid
    col0 = wid * COLS_PER_W

    pltpu.sync_copy(t_hbm.at[pl.ds(col0, COLS_PER_W)], tbuf)

    zeros = jnp.zeros((L,), jnp.float32)
    ones = jnp.full((L,), 1.0, jnp.float32)
    iota = lax.iota(jnp.int32, L)

    def dma(b, r0, nr, buf, sem):
        cstart = pl.multiple_of(col0 + b * BC, BC)
        return pltpu.make_async_copy(
            xt_hbm.at[pl.ds(r0, nr), pl.ds(cstart, BC)],
            buf.at[pl.ds(0, nr)], sem)

    def group_meta(b, g):
        toff = pl.multiple_of(b * BC + g * L, L)
        tg = tbuf[pl.ds(toff, L)]
        validf = lax.select(tg != PAD, ones, zeros)
        return tg, validf

    def proc(buf, nrows, rbase, b, total, accs):
        first = jnp.where(rbase == 0, 1.0, 0.0)
        new_accs = []
        for g in range(GPB):
            tg, validf = group_meta(b, g)

            def dense_step(j, a4, g=g):
                a0, a1, a2, a3 = a4
                r = j * RSTEP
                s = [buf[r + u, pl.ds(g * L, L)] for u in range(RSTEP)]
                return ((a0 + s[0]) + s[4], (a1 + s[1]) + s[5],
                        (a2 + s[2]) + s[6], (a3 + s[3]) + s[7])

            a0, a1, a2, a3 = lax.fori_loop(
                0, nrows // RSTEP, dense_step, (zeros, zeros, zeros, zeros))
            new_accs.append(accs[g] + ((a0 + a1) + (a2 + a3)))

            x00 = buf[0, pl.ds(g * L, L)]
            total = total + first * (validf * (C_ROW + EPS * x00))
        return total, new_accs

    dma(0, 0, SLAB, buf0, sem0).start()

    def batch_body(b, total):
        accs = [zeros] * GPB

        cstart = pl.multiple_of(col0 + b * BC, BC)
        tidx = tbuf.at[pl.ds(pl.multiple_of(b * BC, BC), BC)]
        gather = pltpu.make_async_copy(
            xt_hbm.at[tidx, pl.ds(cstart, BC)], gbuf, semg)
        gather.start()

        def pair_body(p, carry):
            total, *accs = carry
            r0 = p * (2 * SLAB)
            dma(b, r0 + SLAB, SLAB, buf1, sem1).start()
            dma(b, r0, SLAB, buf0, sem0).wait()
            total, accs = proc(buf0, SLAB, r0, b, total, accs)
            dma(b, r0 + 2 * SLAB, SLAB, buf0, sem0).start()
            dma(b, r0 + SLAB, SLAB, buf1, sem1).wait()
            total, accs = proc(buf1, SLAB, r0 + SLAB, b, total, accs)
            return (total, *accs)

        total, *accs = lax.fori_loop(0, 2, pair_body, (total, *accs))

        dma(b, 5 * SLAB, TAILSLAB, buf1, sem1).start()
        dma(b, 4 * SLAB, SLAB, buf0, sem0).wait()
        total, accs = proc(buf0, SLAB, 4 * SLAB, b, total, accs)

        @pl.when(b + 1 < NBATCH)
        def _prefetch():
            dma(b + 1, 0, SLAB, buf0, sem0).start()

        dma(b, 5 * SLAB, TAILSLAB, buf1, sem1).wait()
        total, accs = proc(buf1, TAILSLAB, 5 * SLAB, b, total, accs)

        gather.wait()
        for g in range(GPB):
            _, validf = group_meta(b, g)
            xt_g = zeros
            for l in range(L):
                v = gbuf[g * L + l, pl.ds(g * L, L)]
                xt_g = lax.select(iota == l, v, xt_g)
            total = total - EPS * (validf * accs[g]) \
                - (CONF - EPS) * (validf * xt_g)
        return total

    total = lax.fori_loop(0, NBATCH, batch_body, zeros)
    ovec[...] = total
    pltpu.sync_copy(ovec, out_hbm.at[wid])


@functools.partial(
    pl.kernel,
    mesh=plsc.VectorSubcoreMesh(core_axis_name="c", subcore_axis_name="s"),
    compiler_params=pltpu.CompilerParams(skip_device_barrier=True),
    out_type=jax.ShapeDtypeStruct((NW, L), jnp.float32),
    scratch_types=[
        pltpu.VMEM((SLAB, BC), jnp.float32),
        pltpu.VMEM((SLAB, BC), jnp.float32),
        pltpu.VMEM((COLS_PER_W,), jnp.int32),
        pltpu.VMEM((BC, BC), jnp.float32),
        pltpu.VMEM((L,), jnp.float32),
        pltpu.SemaphoreType.DMA,
        pltpu.SemaphoreType.DMA,
        pltpu.SemaphoreType.DMA,
    ],
)
def _sc_kernel(xt_hbm, t_hbm, out_hbm, buf0, buf1, tbuf, gbuf, ovec,
               sem0, sem1, semg):
    _sc_body(xt_hbm, t_hbm, out_hbm, buf0, buf1, tbuf, gbuf, ovec,
             sem0, sem1, semg)


def _tc_body(xt_ref, tgt_ref, out_ref):
    i = pl.program_id(0)
    x = xt_ref[...]
    t = tgt_ref[...].reshape(1, TCW)
    riota = lax.broadcasted_iota(jnp.int32, (N_COLS, TCW), 0)
    xt_sel = jnp.where(riota == t, x, 0.0)

    colsum = jnp.sum(x, axis=0, keepdims=True)
    xt_val = jnp.sum(xt_sel, axis=0, keepdims=True)
    x0 = x[0:1, :]
    valid = jnp.where(t != PAD, 1.0, 0.0)
    per_col = valid * (C_ROW + EPS * x0 - (CONF - EPS) * xt_val
                       - EPS * colsum)
    part = jnp.sum(per_col.reshape(TGT_ROWS, 128), axis=0, keepdims=True)

    @pl.when(i == 0)
    def _init():
        out_ref[...] = jnp.zeros_like(out_ref)

    out_ref[...] = out_ref[...] + part


_tc_kernel = pl.pallas_call(
    _tc_body,
    grid=(TC_NBLK,),
    in_specs=[
        pl.BlockSpec((N_COLS, TCW), lambda i: (0, SC_COLS // TCW + i)),
        pl.BlockSpec((TCW,), lambda i: (SC_COLS // TCW + i,)),
    ],
    out_specs=pl.BlockSpec((1, 128), lambda i: (0, 0)),
    out_shape=jax.ShapeDtypeStruct((1, 128), jnp.float32),
    compiler_params=pltpu.CompilerParams(
        dimension_semantics=("arbitrary",),
    ),
)


def kernel(x, target):
    xt = x.T
    tgt = target.astype(jnp.int32)
    sc_part = _sc_kernel(xt, tgt)
    tc_part = _tc_kernel(xt, tgt)
    return jnp.sum(sc_part) + jnp.sum(tc_part)


# speedup vs baseline: 1.1260x; 1.0231x over previous
import functools

import numpy as np
import jax
import jax.numpy as jnp
from jax import lax
from jax.experimental import pallas as pl
from jax.experimental.pallas import tpu as pltpu
from jax.experimental.pallas import tpu_sc as plsc

N_ROWS = 16384
N_COLS = 1000
PAD = 0
SMOOTH = 0.1
CONF = 1.0 - SMOOTH
EPS = SMOOTH / (N_COLS - 2)
C_ROW = float(SMOOTH * np.log(EPS) + CONF * np.log(CONF))

NC = 2
NS = 16
L = 16
NW = NC * NS

SC_COLS = 4096
TC_COLS = N_ROWS - SC_COLS
COLS_PER_W = SC_COLS // NW
BC = 128
NBATCH = COLS_PER_W // BC
GPB = BC // L
SLAB = 192
TAILSLAB = N_COLS - 5 * SLAB
RSTEP = 8

TCW = 2048
TC_NBLK = TC_COLS // TCW
TGT_ROWS = TCW // 128


def _sc_body(xt_hbm, t_hbm, out_hbm, buf0, buf1, tbuf, gbuf, ovec,
             sem0, sem1, semg):
    cid = lax.axis_index("c")
    sid = lax.axis_index("s")
    wid = sid * NC + cid
    col0 = wid * COLS_PER_W

    pltpu.sync_copy(t_hbm.at[pl.ds(col0, COLS_PER_W)], tbuf)

    zeros = jnp.zeros((L,), jnp.float32)
    ones = jnp.full((L,), 1.0, jnp.float32)
    iota = lax.iota(jnp.int32, L)

    def dma(b, r0, nr, buf, sem):
        cstart = pl.multiple_of(col0 + b * BC, BC)
        return pltpu.make_async_copy(
            xt_hbm.at[pl.ds(r0, nr), pl.ds(cstart, BC)],
            buf.at[pl.ds(0, nr)], sem)

    def group_meta(b, g):
        toff = pl.multiple_of(b * BC + g * L, L)
        tg = tbuf[pl.ds(toff, L)]
        validf = lax.select(tg != PAD, ones, zeros)
        return tg, validf

    def proc(buf, nrows, rbase, b, total, accs):
        first = jnp.where(rbase == 0, 1.0, 0.0)
        new_accs = []
        for g in range(GPB):
            tg, validf = group_meta(b, g)

            def dense_step(j, a4, g=g):
                a0, a1, a2, a3 = a4
                r = j * RSTEP
                s = [buf[r + u, pl.ds(g * L, L)] for u in range(RSTEP)]
                return ((a0 + s[0]) + s[4], (a1 + s[1]) + s[5],
                        (a2 + s[2]) + s[6], (a3 + s[3]) + s[7])

            a0, a1, a2, a3 = lax.fori_loop(
                0, nrows // RSTEP, dense_step, (zeros, zeros, zeros, zeros))
            new_accs.append(accs[g] + ((a0 + a1) + (a2 + a3)))

            x00 = buf[0, pl.ds(g * L, L)]
            total = total + first * (validf * (C_ROW + EPS * x00))
        return total, new_accs

    dma(0, 0, SLAB, buf0, sem0).start()

    def batch_body(b, total):
        accs = [zeros] * GPB

        cstart = pl.multiple_of(col0 + b * BC, BC)
        tidx = tbuf.at[pl.ds(pl.multiple_of(b * BC, BC), BC)]
        gather = pltpu.make_async_copy(
            xt_hbm.at[tidx, pl.ds(cstart, BC)], gbuf, semg)
        gather.start()

        def pair_body(p, carry):
            total, *accs = carry
            r0 = p * (2 * SLAB)
            dma(b, r0 + SLAB, SLAB, buf1, sem1).start()
            dma(b, r0, SLAB, buf0, sem0).wait()
            total, accs = proc(buf0, SLAB, r0, b, total, accs)
            dma(b, r0 + 2 * SLAB, SLAB, buf0, sem0).start()
            dma(b, r0 + SLAB, SLAB, buf1, sem1).wait()
            total, accs = proc(buf1, SLAB, r0 + SLAB, b, total, accs)
            return (total, *accs)

        total, *accs = lax.fori_loop(0, 2, pair_body, (total, *accs))

        dma(b, 5 * SLAB, TAILSLAB, buf1, sem1).start()
        dma(b, 4 * SLAB, SLAB, buf0, sem0).wait()
        total, accs = proc(buf0, SLAB, 4 * SLAB, b, total, accs)

        @pl.when(b + 1 < NBATCH)
        def _prefetch():
            dma(b + 1, 0, SLAB, buf0, sem0).start()

        dma(b, 5 * SLAB, TAILSLAB, buf1, sem1).wait()
        total, accs = proc(buf1, TAILSLAB, 5 * SLAB, b, total, accs)

        gather.wait()
        for g in range(GPB):
            _, validf = group_meta(b, g)
            xt_g = zeros
            for l in range(L):
                v = gbuf[g * L + l, pl.ds(g * L, L)]
                xt_g = lax.select(iota == l, v, xt_g)
            total = total - EPS * (validf * accs[g]) \
                - (CONF - EPS) * (validf * xt_g)
        return total

    total = lax.fori_loop(0, NBATCH, batch_body, zeros)
    ovec[...] = total
    pltpu.sync_copy(ovec, out_hbm.at[wid])


@functools.partial(
    pl.kernel,
    mesh=plsc.VectorSubcoreMesh(core_axis_name="c", subcore_axis_name="s"),
    compiler_params=pltpu.CompilerParams(skip_device_barrier=True),
    out_type=jax.ShapeDtypeStruct((NW, L), jnp.float32),
    scratch_types=[
        pltpu.VMEM((SLAB, BC), jnp.float32),
        pltpu.VMEM((SLAB, BC), jnp.float32),
        pltpu.VMEM((COLS_PER_W,), jnp.int32),
        pltpu.VMEM((BC, BC), jnp.float32),
        pltpu.VMEM((L,), jnp.float32),
        pltpu.SemaphoreType.DMA,
        pltpu.SemaphoreType.DMA,
        pltpu.SemaphoreType.DMA,
    ],
)
def _sc_kernel(xt_hbm, t_hbm, out_hbm, buf0, buf1, tbuf, gbuf, ovec,
               sem0, sem1, semg):
    _sc_body(xt_hbm, t_hbm, out_hbm, buf0, buf1, tbuf, gbuf, ovec,
             sem0, sem1, semg)


def _tc_body(xt_ref, tgt_ref, out_ref):
    i = pl.program_id(0)
    x = xt_ref[...]
    t = tgt_ref[...].reshape(1, TCW)
    riota = lax.broadcasted_iota(jnp.int32, (N_COLS, TCW), 0)
    xt_sel = jnp.where(riota == t, x, 0.0)

    colsum = jnp.sum(x, axis=0, keepdims=True)
    xt_val = jnp.sum(xt_sel, axis=0, keepdims=True)
    x0 = x[0:1, :]
    valid = jnp.where(t != PAD, 1.0, 0.0)
    per_col = valid * (C_ROW + EPS * x0 - (CONF - EPS) * xt_val
                       - EPS * colsum)
    part = jnp.sum(per_col.reshape(TGT_ROWS, 128), axis=0, keepdims=True)

    @pl.when(i == 0)
    def _init():
        out_ref[...] = jnp.zeros_like(out_ref)

    out_ref[...] = out_ref[...] + part


_tc_kernel = pl.pallas_call(
    _tc_body,
    grid=(TC_NBLK,),
    in_specs=[
        pl.BlockSpec((N_COLS, TCW), lambda i: (0, SC_COLS // TCW + i)),
        pl.BlockSpec((TCW,), lambda i: (SC_COLS // TCW + i,)),
    ],
    out_specs=pl.BlockSpec((1, 128), lambda i: (0, 0)),
    out_shape=jax.ShapeDtypeStruct((1, 128), jnp.float32),
    compiler_params=pltpu.CompilerParams(
        dimension_semantics=("arbitrary",),
    ),
)


def kernel(x, target):
    xt = x.T
    tgt = target.astype(jnp.int32)
    sc_part = _sc_kernel(xt, tgt)
    tc_part = _tc_kernel(xt, tgt)
    return jnp.sum(sc_part) + jnp.sum(tc_part)
